# baseline (speedup 1.0000x reference)
import functools

import numpy as np
import jax
import jax.numpy as jnp
from jax import lax
from jax.experimental import pallas as pl
from jax.experimental.pallas import tpu as pltpu
from jax.experimental.pallas import tpu_sc as plsc

N = 10000
E = 160000
D = 16
H = 4
HD = D // H
NB = 16
RH = 64

NC = 2
NS = 16
NW = NC * NS
EPW = E // NW
IC = 125
KCH = EPW // IC
EP8 = E // 8
NP8 = N // 8

_SC_MESH = dict(core_axis_name="c", subcore_axis_name="s")


def _gather_body(nf_hbm, srcr_hbm, dstr_hbm, xg_hbm, xd_hbm,
                 idx_v, rows_v, sem):
    wid = lax.axis_index("s") * NC + lax.axis_index("c")
    base = wid * EPW
    for idxr_hbm, out_hbm in ((srcr_hbm, xg_hbm), (dstr_hbm, xd_hbm)):
        pltpu.sync_copy(idxr_hbm.at[wid], idx_v)

        @pl.loop(0, KCH, step=8)
        def _grp(r0):
            descs = []
            for j in range(8):
                descs.append(pltpu.async_copy(
                    nf_hbm.at[idx_v.at[r0 + j]],
                    rows_v.at[pl.ds((r0 + j) * IC, IC)], sem))
            for dsc in descs:
                dsc.wait()

        pltpu.sync_copy(rows_v, out_hbm.at[pl.ds(base, EPW)])


def _sc_gather(nf, src_r, dst_r):
    f = functools.partial(
        pl.kernel,
        out_type=(jax.ShapeDtypeStruct((E, D), jnp.float32),
                  jax.ShapeDtypeStruct((E, D), jnp.float32)),
        mesh=plsc.VectorSubcoreMesh(**_SC_MESH),
        compiler_params=pltpu.CompilerParams(use_tc_tiling_on_sc=False),
        scratch_types=[
            pltpu.VMEM((KCH, IC), jnp.int32),
            pltpu.VMEM((EPW, D), jnp.float32),
            pltpu.SemaphoreType.DMA,
        ],
    )(_gather_body)
    return f(nf, src_r, dst_r)


def _scatter_body(wv_hbm, z_hbm, dstr_hbm, out_hbm,
                  idx_v, wv_v, z_v, acc_wv, acc_z):
    c = lax.axis_index("c")
    s = lax.axis_index("s")
    wid = s * NC + c
    base = wid * EPW
    rps = N // NS

    @pl.loop(0, IC)
    def _z(i):
        wv_v[i, pl.ds(0, 16)] = jnp.zeros((16,), jnp.float32)

    for acc in (acc_wv, acc_z):
        for j in range(rps // IC):
            pltpu.sync_copy(wv_v, acc.at[pl.ds(s * rps + j * IC, IC)])
    plsc.subcore_barrier()

    pltpu.sync_copy(dstr_hbm.at[wid], idx_v)

    @pl.loop(0, KCH)
    def _chunk(r):
        pltpu.sync_copy(wv_hbm.at[pl.ds(base + r * IC, IC)], wv_v)
        pltpu.sync_copy(z_hbm.at[pl.ds(base + r * IC, IC)], z_v)
        pltpu.sync_copy(wv_v, acc_wv.at[idx_v.at[r]], add=True)
        pltpu.sync_copy(z_v, acc_z.at[idx_v.at[r]], add=True)

    plsc.subcore_barrier()
    pltpu.sync_copy(acc_wv.at[pl.ds(s * rps, rps)],
                    out_hbm.at[pl.ds(2 * c * N + s * rps, rps)])
    pltpu.sync_copy(acc_z.at[pl.ds(s * rps, rps)],
                    out_hbm.at[pl.ds((2 * c + 1) * N + s * rps, rps)])


def _sc_scatter(wv_lin, z_lin, dst_r):
    f = functools.partial(
        pl.kernel,
        out_type=jax.ShapeDtypeStruct((4 * N, D), jnp.float32),
        mesh=plsc.VectorSubcoreMesh(**_SC_MESH),
        compiler_params=pltpu.CompilerParams(use_tc_tiling_on_sc=False),
        scratch_types=[
            pltpu.VMEM((KCH, IC), jnp.int32),
            pltpu.VMEM((IC, D), jnp.float32),
            pltpu.VMEM((IC, D), jnp.float32),
            pltpu.VMEM_SHARED((N, D), jnp.float32),
            pltpu.VMEM_SHARED((N, D), jnp.float32),
        ],
    )(_scatter_body)
    return f(wv_lin, z_lin, dst_r)


EB = 3200
EBP = EB // 8


def _edge_body(xgp, xdp, remb, shb, elb, scal,
               w1k, b1k, w2k, b2k, w1v, b1v, w2v, b2v,
               wqd, s2r, rbd, sm, owv, oz):
    remb_p = remb[...]
    xs_p = xgp[...] * shb[...]
    hk = jax.nn.silu(remb_p @ w1k[...] + b1k[...])
    hv = jax.nn.silu(remb_p @ w1v[...] + b1v[...])
    xsb = xs_p @ rbd[...]
    ks, vs = [], []
    for j in range(8):
        kw = hk[:, j * RH:(j + 1) * RH] @ w2k[...] + b2k[...]
        vw = hv[:, j * RH:(j + 1) * RH] @ w2v[...] + b2v[...]
        xj = xsb[:, j * 256:(j + 1) * 256]
        ks.append((xj * kw) @ sm[...])
        vs.append((xj * vw) @ sm[...])
    k_p = jnp.concatenate(ks, axis=1)
    v_p = jnp.concatenate(vs, axis=1)
    qw_p = xdp[...] @ wqd[...]
    lb = (qw_p * k_p) @ s2r[...]
    cut = jax.nn.sigmoid(10.0 - elb[...] * scal[...])
    z = jnp.exp(lb * cut)
    owv[...] = v_p * z
    oz[...] = z


def _tc_edges(xgp, xdp, remb, shb, elb, scal, weights):
    full = lambda a: pl.BlockSpec(a.shape, lambda i: (0,) * a.ndim)
    blk = lambda r, w: pl.BlockSpec((r, w), lambda i: (i, 0))
    return pl.pallas_call(
        _edge_body,
        grid=(E // EB,),
        in_specs=[blk(EBP, 128), blk(EBP, 128), blk(EBP, 128),
                  blk(EBP, 128), blk(EBP, 128), full(scal)]
                 + [full(w) for w in weights],
        out_specs=(blk(EBP, 128), blk(EBP, 128)),
        out_shape=(jax.ShapeDtypeStruct((EP8, 128), jnp.float32),
                   jax.ShapeDtypeStruct((EP8, 128), jnp.float32)),
    )(xgp, xdp, remb, shb, elb, scal, *weights)


def _node_body(accp, nfp, wout, wf1, wf2, out):
    a = accp[...]
    wv = a[0:NP8] + a[2 * NP8:3 * NP8]
    z = a[NP8:2 * NP8] + a[3 * NP8:4 * NP8]
    agg = wv / (z + 1e-9)
    y = nfp[...] + agg @ wout[...]
    h1 = y @ wf1[...]
    nrm = jnp.abs(h1)
    act = h1 * (nrm * jax.nn.sigmoid(nrm)) / (nrm + 1e-8)
    out[...] = y + act @ wf2[...]


def _tc_nodes(accp, nfp, wout, wf1, wf2):
    full = lambda a: pl.BlockSpec(a.shape, lambda i: (0,) * a.ndim)
    return pl.pallas_call(
        _node_body,
        grid=(1,),
        in_specs=[full(accp), full(nfp),
                  full(wout), full(wf1), full(wf2)],
        out_specs=full(nfp),
        out_shape=jax.ShapeDtypeStruct((NP8, 128), jnp.float32),
    )(accp, nfp, wout, wf1, wf2)


def _bd8(w):
    return jnp.kron(jnp.eye(8, dtype=jnp.float32), w)


def kernel(node_features, edge_index, edge_sh, edge_radial_emb, edge_length,
           max_radius, W_q, W1k, b1k, W2k, b2k, W1v, b1v, W2v, b2v, W_dot,
           W_out, W_ffn1, W_ffn2):
    f32 = jnp.float32
    src_r = edge_index[0].reshape(NW, KCH, IC)
    dst_r = edge_index[1].reshape(NW, KCH, IC)

    xg, xd = _sc_gather(node_features, src_r, dst_r)
    xgp = xg.reshape(EP8, 128)
    xdp = xd.reshape(EP8, 128)

    sm = jnp.asarray(np.tile(np.eye(D, dtype=np.float32), (D, 1)) * 0.25)
    s2r = _bd8(jnp.asarray(
        np.kron(np.eye(H, dtype=np.float32),
                np.ones((HD, HD), dtype=np.float32)) * 0.25))
    rbd = _bd8(jnp.asarray(
        np.kron(np.eye(D, dtype=np.float32),
                np.ones((1, D), dtype=np.float32))))
    wdbd = jnp.kron(jnp.eye(H, dtype=f32), W_dot)
    wqd = _bd8((W_q * 0.25) @ wdbd)
    w1k = _bd8(W1k)
    w1v = _bd8(W1v)
    b1k8 = jnp.tile(b1k, 8).reshape(1, 8 * RH)
    b1v8 = jnp.tile(b1v, 8).reshape(1, 8 * RH)
    scal = (10.0 / jnp.asarray(max_radius, f32)).reshape(1, 1)
    weights = [w1k, b1k8, W2k, b2k.reshape(1, D * D),
               w1v, b1v8, W2v, b2v.reshape(1, D * D),
               wqd, s2r, rbd, sm]

    shb = jnp.broadcast_to(edge_sh, (E, D)).reshape(EP8, 128)
    elb = jnp.broadcast_to(edge_length, (E, D)).reshape(EP8, 128)
    remb_p = edge_radial_emb.reshape(EP8, 128)
    wv_p, z_p = _tc_edges(xgp, xdp, remb_p, shb, elb, scal, weights)

    acc = _sc_scatter(wv_p.reshape(E, D), z_p.reshape(E, D), dst_r)

    out_p = _tc_nodes(acc.reshape(4 * NP8, 128),
                      node_features.reshape(NP8, 128),
                      _bd8(W_out * 0.25), _bd8(W_ffn1 * 0.25),
                      _bd8(W_ffn2 * (1.0 / np.sqrt(2 * D).astype(np.float32))))
    return out_p.reshape(N, D)


# speedup vs baseline: 1.1054x; 1.1054x over previous
import functools

import numpy as np
import jax
import jax.numpy as jnp
from jax import lax
from jax.experimental import pallas as pl
from jax.experimental.pallas import tpu as pltpu
from jax.experimental.pallas import tpu_sc as plsc

N = 10000
E = 160000
D = 16
H = 4
HD = D // H
NB = 16
RH = 64

NC = 2
NS = 16
NW = NC * NS
EPW = E // NW
IC = 125
KCH = EPW // IC
EP8 = E // 8
NP8 = N // 8

_SC_MESH = dict(core_axis_name="c", subcore_axis_name="s")


def _gather_body(nf_hbm, srcr_hbm, dstr_hbm, xg_hbm, xd_hbm,
                 idx_v, rows_v, sem):
    wid = lax.axis_index("s") * NC + lax.axis_index("c")
    base = wid * EPW
    for idxr_hbm, out_hbm in ((srcr_hbm, xg_hbm), (dstr_hbm, xd_hbm)):
        pltpu.sync_copy(idxr_hbm.at[wid], idx_v)

        @pl.loop(0, KCH, step=8)
        def _grp(r0):
            descs = []
            for j in range(8):
                descs.append(pltpu.async_copy(
                    nf_hbm.at[idx_v.at[r0 + j]],
                    rows_v.at[pl.ds((r0 + j) * IC, IC)], sem))
            for dsc in descs:
                dsc.wait()

        pltpu.sync_copy(rows_v, out_hbm.at[pl.ds(base, EPW)])


def _sc_gather(nf, src_r, dst_r):
    f = functools.partial(
        pl.kernel,
        out_type=(jax.ShapeDtypeStruct((E, D), jnp.float32),
                  jax.ShapeDtypeStruct((E, D), jnp.float32)),
        mesh=plsc.VectorSubcoreMesh(**_SC_MESH),
        compiler_params=pltpu.CompilerParams(use_tc_tiling_on_sc=False),
        scratch_types=[
            pltpu.VMEM((KCH, IC), jnp.int32),
            pltpu.VMEM((EPW, D), jnp.float32),
            pltpu.SemaphoreType.DMA,
        ],
    )(_gather_body)
    return f(nf, src_r, dst_r)


def _scatter_body(wv_hbm, z_hbm, dstr_hbm, out_hbm,
                  idx_v, wv_v, z_v, acc_wv, acc_z):
    c = lax.axis_index("c")
    s = lax.axis_index("s")
    wid = s * NC + c
    base = wid * EPW
    rps = N // NS

    @pl.loop(0, IC)
    def _z(i):
        wv_v[i, pl.ds(0, 16)] = jnp.zeros((16,), jnp.float32)

    for acc in (acc_wv, acc_z):
        for j in range(rps // IC):
            pltpu.sync_copy(wv_v, acc.at[pl.ds(s * rps + j * IC, IC)])
    plsc.subcore_barrier()

    pltpu.sync_copy(dstr_hbm.at[wid], idx_v)

    @pl.loop(0, KCH)
    def _chunk(r):
        pltpu.sync_copy(wv_hbm.at[pl.ds(base + r * IC, IC)], wv_v)
        pltpu.sync_copy(z_hbm.at[pl.ds(base + r * IC, IC)], z_v)
        pltpu.sync_copy(wv_v, acc_wv.at[idx_v.at[r]], add=True)
        pltpu.sync_copy(z_v, acc_z.at[idx_v.at[r]], add=True)

    plsc.subcore_barrier()
    pltpu.sync_copy(acc_wv.at[pl.ds(s * rps, rps)],
                    out_hbm.at[pl.ds(2 * c * N + s * rps, rps)])
    pltpu.sync_copy(acc_z.at[pl.ds(s * rps, rps)],
                    out_hbm.at[pl.ds((2 * c + 1) * N + s * rps, rps)])


def _sc_scatter(wv_lin, z_lin, dst_r):
    f = functools.partial(
        pl.kernel,
        out_type=jax.ShapeDtypeStruct((4 * N, D), jnp.float32),
        mesh=plsc.VectorSubcoreMesh(**_SC_MESH),
        compiler_params=pltpu.CompilerParams(use_tc_tiling_on_sc=False),
        scratch_types=[
            pltpu.VMEM((KCH, IC), jnp.int32),
            pltpu.VMEM((IC, D), jnp.float32),
            pltpu.VMEM((IC, D), jnp.float32),
            pltpu.VMEM_SHARED((N, D), jnp.float32),
            pltpu.VMEM_SHARED((N, D), jnp.float32),
        ],
    )(_scatter_body)
    return f(wv_lin, z_lin, dst_r)


EB = 3200
EBP = EB // 8


def _spread(rows, bsel):
    w = rows @ bsel[...]
    pieces = [w[:, q * 128:(q + 1) * 128] for q in range(16)]
    return jnp.stack(pieces, axis=1).reshape(EBP, 128)


def _edge_body(xgp, xdp, remb, shv, elv, scal,
               w1k, b1k, w2k, b2k, w1v, b1v, w2v, b2v,
               wqd, s2r, rbd, sm, bsel, owv, oz):
    remb_p = remb[...]
    shb = _spread(shv[...].reshape(EB // 128, 128), bsel)
    elb = _spread(elv[...].reshape(EB // 128, 128), bsel)
    xs_p = xgp[...] * shb
    hk = jax.nn.silu(remb_p @ w1k[...] + b1k[...])
    hv = jax.nn.silu(remb_p @ w1v[...] + b1v[...])
    xsb = xs_p @ rbd[...]
    ks, vs = [], []
    for j in range(8):
        kw = hk[:, j * RH:(j + 1) * RH] @ w2k[...] + b2k[...]
        vw = hv[:, j * RH:(j + 1) * RH] @ w2v[...] + b2v[...]
        xj = xsb[:, j * 256:(j + 1) * 256]
        ks.append((xj * kw) @ sm[...])
        vs.append((xj * vw) @ sm[...])
    k_p = jnp.concatenate(ks, axis=1)
    v_p = jnp.concatenate(vs, axis=1)
    qw_p = xdp[...] @ wqd[...]
    lb = (qw_p * k_p) @ s2r[...]
    cut = jax.nn.sigmoid(10.0 - elb * scal[...])
    z = jnp.exp(lb * cut)
    owv[...] = v_p * z
    oz[...] = z


def _tc_edges(xgp, xdp, remb, shv, elv, scal, weights):
    full = lambda a: pl.BlockSpec(a.shape, lambda i: (0,) * a.ndim)
    blk = lambda r, w: pl.BlockSpec((r, w), lambda i: (i, 0))
    sblk = pl.BlockSpec((1, EB // 128, 128), lambda i: (i, 0, 0))
    return pl.pallas_call(
        _edge_body,
        grid=(E // EB,),
        in_specs=[blk(EBP, 128), blk(EBP, 128), blk(EBP, 128),
                  sblk, sblk, full(scal)]
                 + [full(w) for w in weights],
        out_specs=(blk(EBP, 128), blk(EBP, 128)),
        out_shape=(jax.ShapeDtypeStruct((EP8, 128), jnp.float32),
                   jax.ShapeDtypeStruct((EP8, 128), jnp.float32)),
    )(xgp, xdp, remb, shv, elv, scal, *weights)


def _node_body(accp, nfp, wout, wf1, wf2, out):
    a = accp[...]
    wv = a[0:NP8] + a[2 * NP8:3 * NP8]
    z = a[NP8:2 * NP8] + a[3 * NP8:4 * NP8]
    agg = wv / (z + 1e-9)
    y = nfp[...] + agg @ wout[...]
    h1 = y @ wf1[...]
    nrm = jnp.abs(h1)
    act = h1 * (nrm * jax.nn.sigmoid(nrm)) / (nrm + 1e-8)
    out[...] = y + act @ wf2[...]


def _tc_nodes(accp, nfp, wout, wf1, wf2):
    full = lambda a: pl.BlockSpec(a.shape, lambda i: (0,) * a.ndim)
    return pl.pallas_call(
        _node_body,
        grid=(1,),
        in_specs=[full(accp), full(nfp),
                  full(wout), full(wf1), full(wf2)],
        out_specs=full(nfp),
        out_shape=jax.ShapeDtypeStruct((NP8, 128), jnp.float32),
    )(accp, nfp, wout, wf1, wf2)


def _bd8(w):
    return jnp.kron(jnp.eye(8, dtype=jnp.float32), w)


def kernel(node_features, edge_index, edge_sh, edge_radial_emb, edge_length,
           max_radius, W_q, W1k, b1k, W2k, b2k, W1v, b1v, W2v, b2v, W_dot,
           W_out, W_ffn1, W_ffn2):
    f32 = jnp.float32
    src_r = edge_index[0].reshape(NW, KCH, IC)
    dst_r = edge_index[1].reshape(NW, KCH, IC)

    xg, xd = _sc_gather(node_features, src_r, dst_r)
    xgp = xg.reshape(EP8, 128)
    xdp = xd.reshape(EP8, 128)

    sm = jnp.asarray(np.tile(np.eye(D, dtype=np.float32), (D, 1)) * 0.25)
    s2r = _bd8(jnp.asarray(
        np.kron(np.eye(H, dtype=np.float32),
                np.ones((HD, HD), dtype=np.float32)) * 0.25))
    rbd = _bd8(jnp.asarray(
        np.kron(np.eye(D, dtype=np.float32),
                np.ones((1, D), dtype=np.float32))))
    wdbd = jnp.kron(jnp.eye(H, dtype=f32), W_dot)
    wqd = _bd8((W_q * 0.25) @ wdbd)
    w1k = _bd8(W1k)
    w1v = _bd8(W1v)
    b1k8 = jnp.tile(b1k, 8).reshape(1, 8 * RH)
    b1v8 = jnp.tile(b1v, 8).reshape(1, 8 * RH)
    scal = (10.0 / jnp.asarray(max_radius, f32)).reshape(1, 1)
    bsel = np.zeros((128, 2048), dtype=np.float32)
    for q in range(16):
        for p in range(8):
            bsel[8 * q + p, q * 128 + p * 16:q * 128 + p * 16 + 16] = 1.0
    weights = [w1k, b1k8, W2k, b2k.reshape(1, D * D),
               w1v, b1v8, W2v, b2v.reshape(1, D * D),
               wqd, s2r, rbd, sm, jnp.asarray(bsel)]

    shv = edge_sh[:, 0].reshape(E // EB, EB // 128, 128)
    elv = edge_length[:, 0].reshape(E // EB, EB // 128, 128)
    remb_p = edge_radial_emb.reshape(EP8, 128)
    wv_p, z_p = _tc_edges(xgp, xdp, remb_p, shv, elv, scal, weights)

    acc = _sc_scatter(wv_p.reshape(E, D), z_p.reshape(E, D), dst_r)

    out_p = _tc_nodes(acc.reshape(4 * NP8, 128),
                      node_features.reshape(NP8, 128),
                      _bd8(W_out * 0.25), _bd8(W_ffn1 * 0.25),
                      _bd8(W_ffn2 * (1.0 / np.sqrt(2 * D).astype(np.float32))))
    return out_p.reshape(N, D)
